# baseline (speedup 1.0000x reference)
import jax
import jax.numpy as jnp
from jax import lax
from jax.experimental import pallas as pl
from jax.experimental.pallas import tpu as pltpu
from jax.experimental.pallas import tpu_sc as plsc

_S = 24
_N = 577
_D = 64
_NV = _D // 16
_NC = 2
_NS = 16
_NW = _NC * _NS
_RPW = 19
_RW = _N * _D
_TW = (2 * _S + 2) * _D


def _rp2d_body(v_hbm, h_hbm, out_hbm, v_vm, h_vm, row_vm, sem):
    w = lax.axis_index("s") * _NC + lax.axis_index("c")
    pltpu.sync_copy(v_hbm, v_vm)
    pltpu.sync_copy(h_hbm, h_vm)

    cls_vec = [v_vm[pl.ds(d * 16, 16)] + h_vm[pl.ds(d * 16, 16)]
               for d in range(_NV)]

    def wait_one_row():
        pltpu.make_async_copy(
            row_vm.at[pl.ds(0, _RW)], out_hbm.at[pl.ds(0, _RW)], sem).wait()

    def do_row(j, carry):
        q = w + _NW * j
        b = jnp.bitwise_and(j, 1)
        pb = b * _RW

        @pl.when(q < _N)
        def _():
            @pl.when(j >= 2)
            def _():
                wait_one_row()

            @pl.when(q == 0)
            def _():
                def fill(k, p):
                    for d in range(_NV):
                        row_vm[pl.ds(p + 16 * d, 16)] = cls_vec[d]
                    return p + _D
                lax.fori_loop(0, _N, fill, pb)

            @pl.when(q > 0)
            def _():
                qb = (q - 1) // _S
                qr = (q - 1) % _S
                vb = ((_S + 1) - qb) * _D
                hb = ((_S + 1) - qr) * _D
                for d in range(_NV):
                    row_vm[pl.ds(pb + 16 * d, 16)] = cls_vec[d]

                def kb_body(kb, p):
                    vbase = vb + kb * _D
                    vv = [v_vm[pl.ds(vbase + 16 * d, 16)]
                          for d in range(_NV)]
                    for kr in range(_S):
                        ho = hb + kr * _D
                        po = p + kr * _D
                        for d in range(_NV):
                            row_vm[pl.ds(po + 16 * d, 16)] = (
                                vv[d] + h_vm[pl.ds(ho + 16 * d, 16)])
                    return p + _S * _D

                lax.fori_loop(0, _S, kb_body, pb + _D)

            pltpu.async_copy(row_vm.at[pl.ds(pb, _RW)],
                             out_hbm.at[pl.ds(q * _RW, _RW)], sem)

        return carry

    lax.fori_loop(0, _RPW, do_row, 0)
    wait_one_row()
    wait_one_row()


@jax.jit
def _rp2d(table_v, table_h):
    mesh = plsc.VectorSubcoreMesh(
        core_axis_name="c", subcore_axis_name="s",
        num_cores=_NC, num_subcores=_NS)
    out_flat = pl.kernel(
        _rp2d_body,
        out_type=jax.ShapeDtypeStruct((_N * _RW,), jnp.float32),
        mesh=mesh,
        scratch_types=[
            pltpu.VMEM((_TW,), jnp.float32),
            pltpu.VMEM((_TW,), jnp.float32),
            pltpu.VMEM((2 * _RW,), jnp.float32),
            pltpu.SemaphoreType.DMA,
        ],
    )(table_v.reshape(_TW), table_h.reshape(_TW))
    return out_flat.reshape(_N, _N, _D)


def kernel(length_q, length_k, embeddings_table_v, embeddings_table_h):
    del length_q, length_k
    return _rp2d(embeddings_table_v, embeddings_table_h)


# speedup vs baseline: 1.3672x; 1.3672x over previous
import jax
import jax.numpy as jnp
from jax import lax
from jax.experimental import pallas as pl
from jax.experimental.pallas import tpu as pltpu
from jax.experimental.pallas import tpu_sc as plsc

_S = 24
_N = 577
_D = 64
_NV = _D // 16
_NC = 2
_NS = 16
_NW = _NC * _NS
_RPW = 19
_HA = 288


def _rp2d_body(v_hbm, h_hbm, out_hbm, v_vm, h_vm, row_vm, sem_a, sem_b):
    w = lax.axis_index("s") * _NC + lax.axis_index("c")
    pltpu.sync_copy(v_hbm, v_vm)
    pltpu.sync_copy(h_hbm, h_vm)

    cls_vec = [v_vm[0, pl.ds(d * 16, 16)] + h_vm[0, pl.ds(d * 16, 16)]
               for d in range(_NV)]

    def wait_half(sem, lo, n):
        pltpu.make_async_copy(
            row_vm.at[pl.ds(lo, n)], out_hbm.at[0, pl.ds(lo, n)], sem).wait()

    def _slice_starts(q):
        qb = (q - 1) // _S
        qr = (q - 1) % _S
        return (_S + 1) - qb, (_S + 1) - qr

    def _emit_block(vb, hb, kb, kr_lo, kr_hi):
        vv = [v_vm[vb + kb, pl.ds(d * 16, 16)] for d in range(_NV)]
        rbase = 1 + kb * _S
        for kr in range(kr_lo, kr_hi):
            r = rbase + kr
            hrow = hb + kr
            for d in range(_NV):
                row_vm[r, pl.ds(d * 16, 16)] = (
                    vv[d] + h_vm[hrow, pl.ds(d * 16, 16)])

    def build_blocks(vb, hb, kb_lo, kb_hi):
        def kb_body(kb, c):
            _emit_block(vb, hb, kb, 0, _S)
            return c
        lax.fori_loop(kb_lo, kb_hi, kb_body, 0)

    def fill_span(lo, hi):
        def fill(k, c):
            for d in range(_NV):
                row_vm[k, pl.ds(d * 16, 16)] = cls_vec[d]
            return c
        lax.fori_loop(lo, hi, fill, 0)

    def do_row(j, carry):
        q = w + _NW * j

        @pl.when(q < _N)
        def _():
            @pl.when(j >= 1)
            def _():
                wait_half(sem_a, 0, _HA)

            @pl.when(q == 0)
            def _():
                fill_span(0, _HA)

            @pl.when(q > 0)
            def _():
                vb, hb = _slice_starts(q)
                for d in range(_NV):
                    row_vm[0, pl.ds(d * 16, 16)] = cls_vec[d]
                build_blocks(vb, hb, 0, 11)
                _emit_block(vb, hb, 11, 0, _S - 1)

            pltpu.async_copy(row_vm.at[pl.ds(0, _HA)],
                             out_hbm.at[q, pl.ds(0, _HA)], sem_a)

            @pl.when(j >= 1)
            def _():
                wait_half(sem_b, _HA, _N - _HA)

            @pl.when(q == 0)
            def _():
                fill_span(_HA, _N)

            @pl.when(q > 0)
            def _():
                vb, hb = _slice_starts(q)
                _emit_block(vb, hb, 11, _S - 1, _S)
                build_blocks(vb, hb, 12, _S)

            pltpu.async_copy(row_vm.at[pl.ds(_HA, _N - _HA)],
                             out_hbm.at[q, pl.ds(_HA, _N - _HA)], sem_b)

        return carry

    lax.fori_loop(0, _RPW, do_row, 0)
    wait_half(sem_a, 0, _HA)
    wait_half(sem_b, _HA, _N - _HA)


@jax.jit
def _rp2d(table_v, table_h):
    mesh = plsc.VectorSubcoreMesh(
        core_axis_name="c", subcore_axis_name="s",
        num_cores=_NC, num_subcores=_NS)
    return pl.kernel(
        _rp2d_body,
        out_type=jax.ShapeDtypeStruct((_N, _N, _D), jnp.float32),
        mesh=mesh,
        scratch_types=[
            pltpu.VMEM((2 * _S + 2, _D), jnp.float32),
            pltpu.VMEM((2 * _S + 2, _D), jnp.float32),
            pltpu.VMEM((_N, _D), jnp.float32),
            pltpu.SemaphoreType.DMA,
            pltpu.SemaphoreType.DMA,
        ],
    )(table_v, table_h)


def kernel(length_q, length_k, embeddings_table_v, embeddings_table_h):
    del length_q, length_k
    return _rp2d(embeddings_table_v, embeddings_table_h)


# speedup vs baseline: 1.7134x; 1.2532x over previous
import jax
import jax.numpy as jnp
from jax import lax
from jax.experimental import pallas as pl
from jax.experimental.pallas import tpu as pltpu
from jax.experimental.pallas import tpu_sc as plsc

_S = 24
_N = 577
_D = 64
_NV = _D // 16
_NC = 2
_NS = 16
_NW = _NC * _NS
_RPW = 19
_HA = 288


def _rp2d_body(v_hbm, h_hbm, out_hbm, v_vm, h_vm, row_vm, sem_a, sem_b):
    w = lax.axis_index("s") * _NC + lax.axis_index("c")
    pltpu.sync_copy(v_hbm, v_vm)
    pltpu.sync_copy(h_hbm, h_vm)

    cls_vec = [v_vm[0, pl.ds(d * 16, 16)] + h_vm[0, pl.ds(d * 16, 16)]
               for d in range(_NV)]

    def wait_half(sem, lo, n):
        pltpu.make_async_copy(
            row_vm.at[pl.ds(lo, n)], out_hbm.at[0, pl.ds(lo, n)], sem).wait()

    def _slice_starts(q):
        qb = (q - 1) // _S
        qr = (q - 1) % _S
        return (_S + 1) - qb, (_S + 1) - qr

    def _emit_block(vb, hb, kb, kr_lo, kr_hi):
        vv = [v_vm[vb + kb, pl.ds(d * 16, 16)] for d in range(_NV)]
        rbase = 1 + kb * _S
        for kr in range(kr_lo, kr_hi):
            r = rbase + kr
            hrow = hb + kr
            for d in range(_NV):
                row_vm[r, pl.ds(d * 16, 16)] = (
                    vv[d] + h_vm[hrow, pl.ds(d * 16, 16)])

    def build_blocks(vb, hb, kb_lo, kb_hi):
        @plsc.parallel_loop(kb_lo, kb_hi, 1, unroll=2)
        def _(kb):
            _emit_block(vb, hb, kb, 0, _S)

    def fill_span(lo, hi):
        def fill(k, c):
            for d in range(_NV):
                row_vm[k, pl.ds(d * 16, 16)] = cls_vec[d]
            return c
        lax.fori_loop(lo, hi, fill, 0)

    def do_row(j, carry):
        q = w + _NW * j

        @pl.when(q < _N)
        def _():
            @pl.when(j >= 1)
            def _():
                wait_half(sem_a, 0, _HA)

            @pl.when(q == 0)
            def _():
                fill_span(0, _HA)

            @pl.when(q > 0)
            def _():
                vb, hb = _slice_starts(q)
                for d in range(_NV):
                    row_vm[0, pl.ds(d * 16, 16)] = cls_vec[d]
                build_blocks(vb, hb, 0, 11)
                _emit_block(vb, hb, 11, 0, _S - 1)

            pltpu.async_copy(row_vm.at[pl.ds(0, _HA)],
                             out_hbm.at[q, pl.ds(0, _HA)], sem_a)

            @pl.when(j >= 1)
            def _():
                wait_half(sem_b, _HA, _N - _HA)

            @pl.when(q == 0)
            def _():
                fill_span(_HA, _N)

            @pl.when(q > 0)
            def _():
                vb, hb = _slice_starts(q)
                _emit_block(vb, hb, 11, _S - 1, _S)
                build_blocks(vb, hb, 12, _S)

            pltpu.async_copy(row_vm.at[pl.ds(_HA, _N - _HA)],
                             out_hbm.at[q, pl.ds(_HA, _N - _HA)], sem_b)

        return carry

    lax.fori_loop(0, _RPW, do_row, 0)
    wait_half(sem_a, 0, _HA)
    wait_half(sem_b, _HA, _N - _HA)


@jax.jit
def _rp2d(table_v, table_h):
    mesh = plsc.VectorSubcoreMesh(
        core_axis_name="c", subcore_axis_name="s",
        num_cores=_NC, num_subcores=_NS)
    return pl.kernel(
        _rp2d_body,
        out_type=jax.ShapeDtypeStruct((_N, _N, _D), jnp.float32),
        mesh=mesh,
        scratch_types=[
            pltpu.VMEM((2 * _S + 2, _D), jnp.float32),
            pltpu.VMEM((2 * _S + 2, _D), jnp.float32),
            pltpu.VMEM((_N, _D), jnp.float32),
            pltpu.SemaphoreType.DMA,
            pltpu.SemaphoreType.DMA,
        ],
    )(table_v, table_h)


def kernel(length_q, length_k, embeddings_table_v, embeddings_table_h):
    del length_q, length_k
    return _rp2d(embeddings_table_v, embeddings_table_h)
